# speedup vs baseline: 1.2517x; 1.2517x over previous
import functools

import jax
import jax.numpy as jnp
from jax.experimental import pallas as pl
from jax.experimental.pallas import tpu as pltpu

_VMEM_LIMIT_BYTES = 48 * 1024 * 1024


def _se_kernel(x_ref, wr_ref, br_ref, we_ref, be_ref, o_ref, *, nimg, hw):
    inv_hw = 1.0 / float(hw)
    pool_col = jnp.full((hw, 1), inv_hw, dtype=jnp.float32)
    ones_row = jnp.ones((1, hw), dtype=jnp.float32)
    for i in range(nimg):
        x = x_ref[i]
        m = jnp.dot(x, pool_col, preferred_element_type=jnp.float32)
        r = jnp.dot(wr_ref[...], m, preferred_element_type=jnp.float32) + br_ref[...]
        r = r * jax.nn.sigmoid(r)
        e = jnp.dot(we_ref[...], r, preferred_element_type=jnp.float32) + be_ref[...]
        e = jax.nn.sigmoid(e)
        e_bc = jnp.dot(e, ones_row, preferred_element_type=jnp.float32)
        o_ref[i] = (x * e_bc).astype(o_ref.dtype)


def kernel(x_nchw, w_reduce, b_reduce, w_excite, b_excite):
    n, c, h, w = x_nchw.shape
    hw = h * w
    cs = w_reduce.shape[0]

    wr = w_reduce.reshape(cs, c).astype(jnp.float32)
    br = b_reduce.reshape(cs, 1).astype(jnp.float32)
    we = w_excite.reshape(c, cs).astype(jnp.float32)
    be = b_excite.reshape(c, 1).astype(jnp.float32)

    x3 = x_nchw.reshape(n, c, hw)

    nimg = next(d for d in (8, 4, 2, 1) if n % d == 0)
    grid = (n // nimg,)

    y = pl.pallas_call(
        functools.partial(_se_kernel, nimg=nimg, hw=hw),
        out_shape=jax.ShapeDtypeStruct((n, c, hw), x_nchw.dtype),
        grid=grid,
        in_specs=[
            pl.BlockSpec((nimg, c, hw), lambda i: (i, 0, 0)),
            pl.BlockSpec((cs, c), lambda i: (0, 0)),
            pl.BlockSpec((cs, 1), lambda i: (0, 0)),
            pl.BlockSpec((c, cs), lambda i: (0, 0)),
            pl.BlockSpec((c, 1), lambda i: (0, 0)),
        ],
        out_specs=pl.BlockSpec((nimg, c, hw), lambda i: (i, 0, 0)),
        compiler_params=pltpu.CompilerParams(
            dimension_semantics=("parallel",),
            vmem_limit_bytes=_VMEM_LIMIT_BYTES,
        ),
    )(x3, wr, br, we, be)

    return y.reshape(n, c, h, w)


# speedup vs baseline: 7.1540x; 5.7152x over previous
import functools

import jax
import jax.numpy as jnp
from jax.experimental import pallas as pl
from jax.experimental.pallas import tpu as pltpu

_VMEM_LIMIT_BYTES = 48 * 1024 * 1024


def _se_kernel(x_ref, wr_ref, br_ref, we_ref, be_ref, o_ref, *, hw):
    x = x_ref[...]
    m = jnp.sum(x, axis=0) * (1.0 / float(hw))
    r = jnp.dot(m, wr_ref[...], preferred_element_type=jnp.float32) + br_ref[...]
    r = r * jax.nn.sigmoid(r)
    e = jnp.dot(r, we_ref[...], preferred_element_type=jnp.float32) + be_ref[...]
    e = jax.nn.sigmoid(e)
    o_ref[...] = (x * e[None, :, :]).astype(o_ref.dtype)


def kernel(x_nchw, w_reduce, b_reduce, w_excite, b_excite):
    n, c, h, w = x_nchw.shape
    hw = h * w
    cs = w_reduce.shape[0]

    wrT = w_reduce.reshape(cs, c).T.astype(jnp.float32)
    br = b_reduce.reshape(1, cs).astype(jnp.float32)
    weT = w_excite.reshape(c, cs).T.astype(jnp.float32)
    be = b_excite.reshape(1, c).astype(jnp.float32)

    xt = jnp.transpose(x_nchw, (2, 3, 0, 1)).reshape(hw, n, c)

    bn = next(d for d in (16, 8, 4, 2, 1) if n % d == 0)
    grid = (n // bn,)

    y = pl.pallas_call(
        functools.partial(_se_kernel, hw=hw),
        out_shape=jax.ShapeDtypeStruct((hw, n, c), x_nchw.dtype),
        grid=grid,
        in_specs=[
            pl.BlockSpec((hw, bn, c), lambda i: (0, i, 0)),
            pl.BlockSpec((c, cs), lambda i: (0, 0)),
            pl.BlockSpec((1, cs), lambda i: (0, 0)),
            pl.BlockSpec((cs, c), lambda i: (0, 0)),
            pl.BlockSpec((1, c), lambda i: (0, 0)),
        ],
        out_specs=pl.BlockSpec((hw, bn, c), lambda i: (0, i, 0)),
        compiler_params=pltpu.CompilerParams(
            dimension_semantics=("parallel",),
            vmem_limit_bytes=_VMEM_LIMIT_BYTES,
        ),
    )(xt, wrT, br, weT, be)

    return jnp.transpose(y.reshape(h, w, n, c), (2, 3, 0, 1))


# speedup vs baseline: 7.1719x; 1.0025x over previous
import functools

import jax
import jax.numpy as jnp
from jax.experimental import pallas as pl
from jax.experimental.pallas import tpu as pltpu

_VMEM_LIMIT_BYTES = 48 * 1024 * 1024


_TRANS_B = (((1,), (1,)), ((), ()))


def _se_kernel(x_ref, wr_ref, br_ref, we_ref, be_ref, o_ref, *, hw):
    x = x_ref[...]
    m = jnp.sum(x, axis=0) * (1.0 / float(hw))
    r = jax.lax.dot_general(m, wr_ref[...], _TRANS_B,
                            preferred_element_type=jnp.float32) + br_ref[...]
    r = r * jax.nn.sigmoid(r)
    e = jax.lax.dot_general(r, we_ref[...], _TRANS_B,
                            preferred_element_type=jnp.float32) + be_ref[...]
    e = jax.nn.sigmoid(e)
    o_ref[...] = (x * e[None, :, :]).astype(o_ref.dtype)


def kernel(x_nchw, w_reduce, b_reduce, w_excite, b_excite):
    n, c, h, w = x_nchw.shape
    hw = h * w
    cs = w_reduce.shape[0]

    wr = w_reduce.reshape(cs, c).astype(jnp.float32)
    br = b_reduce.reshape(1, cs).astype(jnp.float32)
    we = w_excite.reshape(c, cs).astype(jnp.float32)
    be = b_excite.reshape(1, c).astype(jnp.float32)

    xt = jnp.transpose(x_nchw, (2, 3, 0, 1)).reshape(hw, n, c)

    bn = next(d for d in (16, 8, 4, 2, 1) if n % d == 0)
    grid = (n // bn,)

    y = pl.pallas_call(
        functools.partial(_se_kernel, hw=hw),
        out_shape=jax.ShapeDtypeStruct((hw, n, c), x_nchw.dtype),
        grid=grid,
        in_specs=[
            pl.BlockSpec((hw, bn, c), lambda i: (0, i, 0)),
            pl.BlockSpec((cs, c), lambda i: (0, 0)),
            pl.BlockSpec((1, cs), lambda i: (0, 0)),
            pl.BlockSpec((c, cs), lambda i: (0, 0)),
            pl.BlockSpec((1, c), lambda i: (0, 0)),
        ],
        out_specs=pl.BlockSpec((hw, bn, c), lambda i: (0, i, 0)),
        compiler_params=pltpu.CompilerParams(
            dimension_semantics=("parallel",),
            vmem_limit_bytes=_VMEM_LIMIT_BYTES,
        ),
    )(xt, wr, br, we, be)

    return jnp.transpose(y.reshape(h, w, n, c), (2, 3, 0, 1))
